# baseline (speedup 1.0000x reference)
import functools

import jax
import jax.numpy as jnp
from jax import lax
from jax.experimental import pallas as pl
from jax.experimental.pallas import tpu as pltpu
from jax.experimental.pallas import tpu_sc as plsc

N = 10000
E = 320000
D = 128
NC, NS = 2, 16
NW = NC * NS
CHUNK = 128
CPT = 80
E_PAD = CPT * CHUNK * NW
NBUF = 4
ROWS_PER_TILE = 640
N_ACC = ROWS_PER_TILE * NS

_mesh = plsc.VectorSubcoreMesh(core_axis_name="c", subcore_axis_name="s")


def _deg_body(col_hbm, zerosd_hbm, ones_hbm, degp_hbm,
              deg_sh, cidx_all, ones_v, ssem):
    c = lax.axis_index("c")
    s = lax.axis_index("s")
    wid = s * NC + c
    base = s * ROWS_PER_TILE
    pltpu.sync_copy(col_hbm.at[wid], cidx_all)
    pltpu.sync_copy(ones_hbm, ones_v)
    pltpu.sync_copy(zerosd_hbm, deg_sh.at[pl.ds(base, ROWS_PER_TILE)])
    plsc.subcore_barrier()

    def fire(i, carry):
        pltpu.async_copy(ones_v, deg_sh.at[cidx_all.at[i]], ssem, add=True)
        return carry

    lax.fori_loop(0, CPT, fire, 0)

    def drain(i, carry):
        pltpu.make_async_copy(zerosd_hbm.at[pl.ds(0, CHUNK)], ones_v, ssem).wait()
        return carry

    lax.fori_loop(0, CPT, drain, 0)
    plsc.subcore_barrier()
    pltpu.sync_copy(deg_sh.at[pl.ds(base, ROWS_PER_TILE)],
                    degp_hbm.at[c, pl.ds(base, ROWS_PER_TILE)])


_deg_kernel = functools.partial(
    pl.kernel,
    out_type=jax.ShapeDtypeStruct((NC, N_ACC, D), jnp.float32),
    mesh=_mesh,
    scratch_types=[
        pltpu.VMEM_SHARED((N_ACC, D), jnp.float32),
        pltpu.VMEM((CPT, CHUNK), jnp.int32),
        pltpu.VMEM((CHUNK, D), jnp.float32),
        pltpu.SemaphoreType.DMA,
    ],
)(_deg_body)


SB = 16
NSB = CPT // SB


def _agg_body(row_hbm, col_hbm, y_hbm, zerosd_hbm, out_hbm,
              acc_sh, ridx, cidx, ybufs, isems, gsems, ssems):
    c = lax.axis_index("c")
    s = lax.axis_index("s")
    wid = s * NC + c
    base = s * ROWS_PER_TILE

    def fetch_block(b):
        slot = b % 2
        pltpu.async_copy(row_hbm.at[wid, pl.ds(b * SB, SB)],
                         ridx.at[slot], isems.at[slot])
        pltpu.async_copy(col_hbm.at[wid, pl.ds(b * SB, SB)],
                         cidx.at[slot], isems.at[slot])

    def wait_idx(slot):
        for _ in range(2):
            pltpu.make_async_copy(row_hbm.at[wid, pl.ds(0, SB)],
                                  ridx.at[slot], isems.at[slot]).wait()

    def wait_data(sems, j):
        pltpu.make_async_copy(y_hbm.at[pl.ds(0, CHUNK)], ybufs.at[j],
                              sems.at[j]).wait()

    fetch_block(0)
    fetch_block(1)
    pltpu.sync_copy(zerosd_hbm, acc_sh.at[pl.ds(base, ROWS_PER_TILE)])
    plsc.subcore_barrier()

    for so in range(NSB):
        slot = so % 2
        wait_idx(slot)

        def step(k2, carry):
            for j in range(2):
                ch = 2 * k2 + j

                @pl.when(k2 >= 1)
                def _():
                    wait_data(ssems, j)
                pltpu.async_copy(y_hbm.at[ridx.at[slot, ch]],
                                 ybufs.at[j], gsems.at[j])
                if j == 1:
                    wait_data(gsems, 0)
                    pltpu.async_copy(ybufs.at[0], acc_sh.at[cidx.at[slot, ch - 1]],
                                     ssems.at[0], add=True)
                else:
                    @pl.when(k2 >= 1)
                    def _():
                        wait_data(gsems, 1)
                        pltpu.async_copy(
                            ybufs.at[1], acc_sh.at[cidx.at[slot, 2 * k2 - 1]],
                            ssems.at[1], add=True)
            return carry

        lax.fori_loop(0, SB // 2, step, 0)
        wait_data(gsems, 1)
        pltpu.async_copy(ybufs.at[1], acc_sh.at[cidx.at[slot, SB - 1]],
                         ssems.at[1], add=True)
        wait_data(ssems, 0)
        wait_data(ssems, 1)
        if so + 2 < NSB:
            fetch_block(so + 2)
    plsc.subcore_barrier()
    pltpu.sync_copy(acc_sh.at[pl.ds(base, ROWS_PER_TILE)],
                    out_hbm.at[c, pl.ds(base, ROWS_PER_TILE)])


_agg_kernel = functools.partial(
    pl.kernel,
    out_type=jax.ShapeDtypeStruct((NC, N_ACC, D), jnp.float32),
    mesh=_mesh,
    scratch_types=[
        pltpu.VMEM_SHARED((N_ACC, D), jnp.float32),
        pltpu.VMEM((2, SB, CHUNK), jnp.int32),
        pltpu.VMEM((2, SB, CHUNK), jnp.int32),
        pltpu.VMEM((2, CHUNK, D), jnp.float32),
        pltpu.SemaphoreType.DMA((2,)),
        pltpu.SemaphoreType.DMA((2,)),
        pltpu.SemaphoreType.DMA((2,)),
    ],
)(_agg_body)


_BLK = 1000


def _dis(d0_ref, d1_ref):
    deg = d0_ref[...] + d1_ref[...] + 1.0
    return lax.rsqrt(deg)


def _mm1_body(x_ref, w_ref, d0_ref, d1_ref, y_ref):
    y_ref[...] = _dis(d0_ref, d1_ref) * jnp.dot(
        x_ref[...], w_ref[...], preferred_element_type=jnp.float32)


def _mid_body(a0_ref, a1_ref, y_ref, w_ref, b_ref, d0_ref, d1_ref, o_ref):
    dis = _dis(d0_ref, d1_ref)
    h = dis * (a0_ref[...] + a1_ref[...] + y_ref[...]) + b_ref[...]
    h = jnp.maximum(h, 0.0)
    o_ref[...] = dis * jnp.dot(h, w_ref[...], preferred_element_type=jnp.float32)


def _fin_body(a0_ref, a1_ref, y_ref, b_ref, d0_ref, d1_ref, o_ref):
    dis = _dis(d0_ref, d1_ref)
    o_ref[...] = dis * (a0_ref[...] + a1_ref[...] + y_ref[...]) + b_ref[...]


def _row_spec(width):
    return pl.BlockSpec((_BLK, width), lambda i: (i, 0))


def _full_spec(shape):
    return pl.BlockSpec(shape, lambda i: (0, 0))


def _mm1(x, w, d0, d1):
    return pl.pallas_call(
        _mm1_body,
        grid=(N // _BLK,),
        in_specs=[_row_spec(D), _full_spec((D, D)), _row_spec(1), _row_spec(1)],
        out_specs=_row_spec(D),
        out_shape=jax.ShapeDtypeStruct((N, D), jnp.float32),
    )(x, w, d0, d1)


def _mid(a0, a1, y, w, b, d0, d1):
    return pl.pallas_call(
        _mid_body,
        grid=(N // _BLK,),
        in_specs=[_row_spec(D), _row_spec(D), _row_spec(D), _full_spec((D, D)),
                  _full_spec((1, D)), _row_spec(1), _row_spec(1)],
        out_specs=_row_spec(D),
        out_shape=jax.ShapeDtypeStruct((N, D), jnp.float32),
    )(a0, a1, y, w, b, d0, d1)


def _fin(a0, a1, y, b, d0, d1):
    return pl.pallas_call(
        _fin_body,
        grid=(N // _BLK,),
        in_specs=[_row_spec(D), _row_spec(D), _row_spec(D),
                  _full_spec((1, D)), _row_spec(1), _row_spec(1)],
        out_specs=_row_spec(D),
        out_shape=jax.ShapeDtypeStruct((N, D), jnp.float32),
    )(a0, a1, y, b, d0, d1)


def kernel(x, edge_index, W1, b1, W2, b2, W3, b3):
    ei = edge_index.astype(jnp.int32)
    pad_r = jnp.zeros((E_PAD - E,), jnp.int32)
    pad_c = N + (jnp.arange(E_PAD - E, dtype=jnp.int32) % (N_ACC - N))
    row_t = jnp.concatenate([ei[0], pad_r]).reshape(NW, CPT, CHUNK)
    col_t = jnp.concatenate([ei[1], pad_c]).reshape(NW, CPT, CHUNK)

    onesd = jnp.ones((CHUNK, D), jnp.float32)
    zerosd = jnp.zeros((ROWS_PER_TILE, D), jnp.float32)

    degp = _deg_kernel(col_t, zerosd, onesd)
    d0 = degp[0, :N, 0:1]
    d1 = degp[1, :N, 0:1]

    b1r = b1.reshape(1, D)
    b2r = b2.reshape(1, D)
    b3r = b3.reshape(1, D)

    y1 = _mm1(x, W1, d0, d1)
    a = _agg_kernel(row_t, col_t, y1, zerosd)
    y2 = _mid(a[0, :N], a[1, :N], y1, W2, b1r, d0, d1)
    a = _agg_kernel(row_t, col_t, y2, zerosd)
    y3 = _mid(a[0, :N], a[1, :N], y2, W3, b2r, d0, d1)
    a = _agg_kernel(row_t, col_t, y3, zerosd)
    return _fin(a[0, :N], a[1, :N], y3, b3r, d0, d1)


# speedup vs baseline: 3.2395x; 3.2395x over previous
import functools

import jax
import jax.numpy as jnp
from jax import lax
from jax.experimental import pallas as pl
from jax.experimental.pallas import tpu as pltpu
from jax.experimental.pallas import tpu_sc as plsc

N = 10000
E = 320000
D = 128
NC, NS = 2, 16
NW = NC * NS
CHUNK = 128
CPT = 80
E_PAD = CPT * CHUNK * NW
NBUF = 4
ROWS_PER_TILE = 640
N_ACC = ROWS_PER_TILE * NS

_mesh = plsc.VectorSubcoreMesh(core_axis_name="c", subcore_axis_name="s")


def _deg_body(col_hbm, zerosd_hbm, ones_hbm, degp_hbm,
              deg_sh, cidx_all, ones_v, ssem):
    c = lax.axis_index("c")
    s = lax.axis_index("s")
    wid = s * NC + c
    base = s * ROWS_PER_TILE
    pltpu.sync_copy(col_hbm.at[wid], cidx_all)
    pltpu.sync_copy(ones_hbm, ones_v)
    pltpu.sync_copy(zerosd_hbm, deg_sh.at[pl.ds(base, ROWS_PER_TILE)])
    plsc.subcore_barrier()

    def fire(i, carry):
        pltpu.async_copy(ones_v, deg_sh.at[cidx_all.at[i]], ssem, add=True)
        return carry

    lax.fori_loop(0, CPT, fire, 0)

    def drain(i, carry):
        pltpu.make_async_copy(zerosd_hbm.at[pl.ds(0, CHUNK)], ones_v, ssem).wait()
        return carry

    lax.fori_loop(0, CPT, drain, 0)
    plsc.subcore_barrier()
    pltpu.sync_copy(deg_sh.at[pl.ds(base, ROWS_PER_TILE)],
                    degp_hbm.at[c, pl.ds(base, ROWS_PER_TILE)])


_deg_kernel = functools.partial(
    pl.kernel,
    out_type=jax.ShapeDtypeStruct((NC, N_ACC, D), jnp.float32),
    mesh=_mesh,
    scratch_types=[
        pltpu.VMEM_SHARED((N_ACC, D), jnp.float32),
        pltpu.VMEM((CPT, CHUNK), jnp.int32),
        pltpu.VMEM((CHUNK, D), jnp.float32),
        pltpu.SemaphoreType.DMA,
    ],
)(_deg_body)


SB = 16
NSB = CPT // SB


def _agg_body(row_hbm, col_hbm, y_hbm, zerosd_hbm, out_hbm,
              acc_sh, ridx, cidx, ybufs, isems, gsems, ssems):
    c = lax.axis_index("c")
    s = lax.axis_index("s")
    wid = s * NC + c
    base = s * ROWS_PER_TILE

    def fetch_block(b):
        slot = b % 2
        pltpu.async_copy(row_hbm.at[wid, pl.ds(b * SB, SB)],
                         ridx.at[slot], isems.at[slot])
        pltpu.async_copy(col_hbm.at[wid, pl.ds(b * SB, SB)],
                         cidx.at[slot], isems.at[slot])

    def wait_idx(slot):
        for _ in range(2):
            pltpu.make_async_copy(row_hbm.at[wid, pl.ds(0, SB)],
                                  ridx.at[slot], isems.at[slot]).wait()

    def wait_data(sems, j):
        pltpu.make_async_copy(y_hbm.at[pl.ds(0, CHUNK)], ybufs.at[j],
                              sems.at[j]).wait()

    fetch_block(0)
    fetch_block(1)
    pltpu.sync_copy(zerosd_hbm, acc_sh.at[pl.ds(base, ROWS_PER_TILE)])
    plsc.subcore_barrier()

    for so in range(NSB):
        slot = so % 2
        wait_idx(slot)

        def step(k2, carry):
            for j in range(2):
                ch = 2 * k2 + j

                @pl.when(k2 >= 1)
                def _():
                    wait_data(ssems, j)
                pltpu.async_copy(y_hbm.at[ridx.at[slot, ch]],
                                 ybufs.at[j], gsems.at[j])
                if j == 1:
                    wait_data(gsems, 0)
                    pltpu.async_copy(ybufs.at[0], acc_sh.at[cidx.at[slot, ch - 1]],
                                     ssems.at[0], add=True)
                else:
                    @pl.when(k2 >= 1)
                    def _():
                        wait_data(gsems, 1)
                        pltpu.async_copy(
                            ybufs.at[1], acc_sh.at[cidx.at[slot, 2 * k2 - 1]],
                            ssems.at[1], add=True)
            return carry

        lax.fori_loop(0, SB // 2, step, 0)
        wait_data(gsems, 1)
        pltpu.async_copy(ybufs.at[1], acc_sh.at[cidx.at[slot, SB - 1]],
                         ssems.at[1], add=True)
        wait_data(ssems, 0)
        wait_data(ssems, 1)
        if so + 2 < NSB:
            fetch_block(so + 2)
    plsc.subcore_barrier()
    pltpu.sync_copy(acc_sh.at[pl.ds(base, ROWS_PER_TILE)],
                    out_hbm.at[c, pl.ds(base, ROWS_PER_TILE)])


_agg_kernel = functools.partial(
    pl.kernel,
    out_type=jax.ShapeDtypeStruct((NC, N_ACC, D), jnp.float32),
    mesh=_mesh,
    scratch_types=[
        pltpu.VMEM_SHARED((N_ACC, D), jnp.float32),
        pltpu.VMEM((2, SB, CHUNK), jnp.int32),
        pltpu.VMEM((2, SB, CHUNK), jnp.int32),
        pltpu.VMEM((2, CHUNK, D), jnp.float32),
        pltpu.SemaphoreType.DMA((2,)),
        pltpu.SemaphoreType.DMA((2,)),
        pltpu.SemaphoreType.DMA((2,)),
    ],
)(_agg_body)


_BLK = 1000


def _dis(d0_ref, d1_ref):
    deg = d0_ref[...] + d1_ref[...] + 1.0
    return lax.rsqrt(deg)


def _mm1_body(x_ref, w_ref, d0_ref, d1_ref, y_ref):
    y_ref[...] = _dis(d0_ref, d1_ref) * jnp.dot(
        x_ref[...], w_ref[...], preferred_element_type=jnp.float32)


def _mid_body(a0_ref, a1_ref, y_ref, w_ref, b_ref, d0_ref, d1_ref, o_ref):
    dis = _dis(d0_ref, d1_ref)
    h = dis * (a0_ref[...] + a1_ref[...] + y_ref[...]) + b_ref[...]
    h = jnp.maximum(h, 0.0)
    o_ref[...] = dis * jnp.dot(h, w_ref[...], preferred_element_type=jnp.float32)


def _fin_body(a0_ref, a1_ref, y_ref, b_ref, d0_ref, d1_ref, o_ref):
    dis = _dis(d0_ref, d1_ref)
    o_ref[...] = dis * (a0_ref[...] + a1_ref[...] + y_ref[...]) + b_ref[...]


def _row_spec(width):
    return pl.BlockSpec((_BLK, width), lambda i: (i, 0))


def _full_spec(shape):
    return pl.BlockSpec(shape, lambda i: (0, 0))


def _mm1(x, w, d0, d1):
    return pl.pallas_call(
        _mm1_body,
        grid=(N // _BLK,),
        in_specs=[_row_spec(D), _full_spec((D, D)), _row_spec(1), _row_spec(1)],
        out_specs=_row_spec(D),
        out_shape=jax.ShapeDtypeStruct((N, D), jnp.float32),
    )(x, w, d0, d1)


def _mid(a0, a1, y, w, b, d0, d1):
    return pl.pallas_call(
        _mid_body,
        grid=(N // _BLK,),
        in_specs=[_row_spec(D), _row_spec(D), _row_spec(D), _full_spec((D, D)),
                  _full_spec((1, D)), _row_spec(1), _row_spec(1)],
        out_specs=_row_spec(D),
        out_shape=jax.ShapeDtypeStruct((N, D), jnp.float32),
    )(a0, a1, y, w, b, d0, d1)


def _fin(a0, a1, y, b, d0, d1):
    return pl.pallas_call(
        _fin_body,
        grid=(N // _BLK,),
        in_specs=[_row_spec(D), _row_spec(D), _row_spec(D),
                  _full_spec((1, D)), _row_spec(1), _row_spec(1)],
        out_specs=_row_spec(D),
        out_shape=jax.ShapeDtypeStruct((N, D), jnp.float32),
    )(a0, a1, y, b, d0, d1)


def kernel(x, edge_index, W1, b1, W2, b2, W3, b3):
    ei = edge_index.astype(jnp.int32)
    pad_r = jnp.arange(E_PAD - E, dtype=jnp.int32) % N
    pad_c = N + (jnp.arange(E_PAD - E, dtype=jnp.int32) % (N_ACC - N))
    row_t = jnp.concatenate([ei[0], pad_r]).reshape(NW, CPT, CHUNK)
    col_t = jnp.concatenate([ei[1], pad_c]).reshape(NW, CPT, CHUNK)

    onesd = jnp.ones((CHUNK, D), jnp.float32)
    zerosd = jnp.zeros((ROWS_PER_TILE, D), jnp.float32)

    degp = _deg_kernel(col_t, zerosd, onesd)
    d0 = degp[0, :N, 0:1]
    d1 = degp[1, :N, 0:1]

    b1r = b1.reshape(1, D)
    b2r = b2.reshape(1, D)
    b3r = b3.reshape(1, D)

    y1 = _mm1(x, W1, d0, d1)
    a = _agg_kernel(row_t, col_t, y1, zerosd)
    y2 = _mid(a[0, :N], a[1, :N], y1, W2, b1r, d0, d1)
    a = _agg_kernel(row_t, col_t, y2, zerosd)
    y3 = _mid(a[0, :N], a[1, :N], y2, W3, b2r, d0, d1)
    a = _agg_kernel(row_t, col_t, y3, zerosd)
    return _fin(a[0, :N], a[1, :N], y3, b3r, d0, d1)
